# Initial kernel scaffold; baseline (speedup 1.0000x reference)
import functools

import jax
import jax.numpy as jnp
from jax import lax
from jax.experimental import pallas as pl
from jax.experimental.pallas import tpu as pltpu
from jax.experimental.pallas import tpu_sc as plsc

N_NODES = 10000
D = 128
N_REL = 8
E = 320000
R_CAT = N_REL + 1

NC, NS = 2, 16
NW = NC * NS
EPW = E // NW
CHUNK = 80
NCH = EPW // CHUNK
ROWS_PT = N_NODES // NS

_MESH = plsc.VectorSubcoreMesh(core_axis_name="c", subcore_axis_name="s")


def _mm_body(x_ref, w_ref, o_ref):
    o_ref[0] = jnp.dot(x_ref[...], w_ref[0], preferred_element_type=jnp.float32)


def _tc_matmul(x, wcat, nb=5):
    blk = N_NODES // nb
    return pl.pallas_call(
        _mm_body,
        grid=(R_CAT, nb),
        in_specs=[
            pl.BlockSpec((blk, D), lambda r, b: (b, 0)),
            pl.BlockSpec((1, D, D), lambda r, b: (r, 0, 0)),
        ],
        out_specs=pl.BlockSpec((1, blk, D), lambda r, b: (r, b, 0)),
        out_shape=jax.ShapeDtypeStruct((R_CAT, N_NODES, D), jnp.float32),
    )(x, wcat)


def _sc_body(hall, gidx_hbm, dst_hbm, zeros_hbm, out_hbm,
             gidx_v, dst_v, rows_v, agg_sp, sem):
    c = lax.axis_index("c")
    s = lax.axis_index("s")
    wid = s * NC + c

    pltpu.sync_copy(gidx_hbm.at[wid], gidx_v)
    pltpu.sync_copy(dst_hbm.at[wid], dst_v)
    pltpu.sync_copy(zeros_hbm.at[pl.ds(s * ROWS_PT, ROWS_PT)],
                    agg_sp.at[pl.ds(s * ROWS_PT, ROWS_PT)])
    plsc.subcore_barrier()

    def body(j, carry):
        pltpu.async_copy(hall.at[gidx_v.at[j]], rows_v, sem).wait()
        pltpu.sync_copy(rows_v, agg_sp.at[dst_v.at[j]], add=True)
        return carry

    lax.fori_loop(0, NCH, body, 0)
    plsc.subcore_barrier()
    pltpu.sync_copy(agg_sp.at[pl.ds(s * ROWS_PT, ROWS_PT)],
                    out_hbm.at[c, pl.ds(s * ROWS_PT, ROWS_PT)])


_sc_aggregate = functools.partial(
    pl.kernel,
    out_type=jax.ShapeDtypeStruct((NC, N_NODES, D), jnp.float32),
    mesh=_MESH,
    scratch_types=[
        pltpu.VMEM((NCH, CHUNK), jnp.int32),
        pltpu.VMEM((NCH, CHUNK), jnp.int32),
        pltpu.VMEM((CHUNK, D), jnp.float32),
        pltpu.VMEM_SHARED((N_NODES, D), jnp.float32),
        pltpu.SemaphoreType.DMA,
    ],
)(_sc_body)


def _combine_body(act, p_ref, xr_ref, b_ref, o_ref):
    o_ref[...] = act(p_ref[0] + p_ref[1] + xr_ref[0] + b_ref[0][None, :])


def _tc_combine(p, hall, b, act, nb=5):
    blk = N_NODES // nb
    return pl.pallas_call(
        functools.partial(_combine_body, act),
        grid=(nb,),
        in_specs=[
            pl.BlockSpec((NC, blk, D), lambda i: (0, i, 0)),
            pl.BlockSpec((1, blk, D), lambda i: (R_CAT - 1, i, 0)),
            pl.BlockSpec((1, D), lambda i: (0, 0)),
        ],
        out_specs=pl.BlockSpec((blk, D), lambda i: (i, 0)),
        out_shape=jax.ShapeDtypeStruct((N_NODES, D), jnp.float32),
    )(p, hall, b)


def _layer(x, wcat, b, gidx, dsti, zeros, act):
    hall = _tc_matmul(x, wcat)
    p = _sc_aggregate(hall.reshape(R_CAT * N_NODES, D), gidx, dsti, zeros)
    return _tc_combine(p, hall, b.reshape(1, D), act)


def kernel(x, edge_index, edge_type, W1, root1, b1, W2, root2, b2):
    src = edge_index[0].astype(jnp.int32)
    dst = edge_index[1].astype(jnp.int32)
    et = edge_type.astype(jnp.int32)
    gidx = (et * N_NODES + src).reshape(NW, NCH, CHUNK)
    dsti = dst.reshape(NW, NCH, CHUNK)
    zeros = jnp.zeros((N_NODES, D), jnp.float32)
    wcat1 = jnp.concatenate([W1, root1[None]], axis=0)
    wcat2 = jnp.concatenate([W2, root2[None]], axis=0)
    h = _layer(x, wcat1, b1, gidx, dsti, zeros,
               lambda v: jnp.maximum(v, 0.0))
    return _layer(h, wcat2, b2, gidx, dsti, zeros, jax.nn.sigmoid)


# speedup vs baseline: 21.2744x; 21.2744x over previous
import functools

import jax
import jax.numpy as jnp
from jax import lax
from jax.experimental import pallas as pl
from jax.experimental.pallas import tpu as pltpu
from jax.experimental.pallas import tpu_sc as plsc

N_NODES = 10000
D = 128
N_REL = 8
E = 320000
R_CAT = N_REL + 1

NC, NS = 2, 16
NW = NC * NS
EPW = E // NW
CHUNK = 80
NCH = EPW // CHUNK
ROWS_PT = 624
ROWS_TAIL = N_NODES - NS * ROWS_PT

_MESH = plsc.VectorSubcoreMesh(core_axis_name="c", subcore_axis_name="s")


def _mm_body(x_ref, w_ref, o_ref):
    o_ref[0] = jnp.dot(x_ref[...], w_ref[0], preferred_element_type=jnp.float32)


def _tc_matmul(x, wcat, nb=5):
    blk = N_NODES // nb
    return pl.pallas_call(
        _mm_body,
        grid=(R_CAT, nb),
        in_specs=[
            pl.BlockSpec((blk, D), lambda r, b: (b, 0)),
            pl.BlockSpec((1, D, D), lambda r, b: (r, 0, 0)),
        ],
        out_specs=pl.BlockSpec((1, blk, D), lambda r, b: (r, b, 0)),
        out_shape=jax.ShapeDtypeStruct((R_CAT, N_NODES, D), jnp.float32),
    )(x, wcat)


def _sc_body(hall, gidx_hbm, dst_hbm, zeros_hbm, out_hbm,
             gidx_v, dst_v, rows_v, agg_sp, sem):
    c = lax.axis_index("c")
    s = lax.axis_index("s")
    wid = s * NC + c

    pltpu.sync_copy(gidx_hbm.at[wid], gidx_v)
    pltpu.sync_copy(dst_hbm.at[wid], dst_v)
    pltpu.sync_copy(zeros_hbm.at[pl.ds(s * ROWS_PT, ROWS_PT)],
                    agg_sp.at[pl.ds(s * ROWS_PT, ROWS_PT)])

    @pl.when(s == 0)
    def _init_tail():
        pltpu.sync_copy(zeros_hbm.at[pl.ds(NS * ROWS_PT, ROWS_TAIL)],
                        agg_sp.at[pl.ds(NS * ROWS_PT, ROWS_TAIL)])

    plsc.subcore_barrier()

    def body(j, carry):
        pltpu.async_copy(hall.at[gidx_v.at[j]], rows_v, sem).wait()
        pltpu.sync_copy(rows_v, agg_sp.at[dst_v.at[j]], add=True)
        return carry

    lax.fori_loop(0, NCH, body, 0)
    plsc.subcore_barrier()
    pltpu.sync_copy(agg_sp.at[pl.ds(s * ROWS_PT, ROWS_PT)],
                    out_hbm.at[c, pl.ds(s * ROWS_PT, ROWS_PT)])

    @pl.when(s == 0)
    def _write_tail():
        pltpu.sync_copy(agg_sp.at[pl.ds(NS * ROWS_PT, ROWS_TAIL)],
                        out_hbm.at[c, pl.ds(NS * ROWS_PT, ROWS_TAIL)])


_sc_aggregate = functools.partial(
    pl.kernel,
    out_type=jax.ShapeDtypeStruct((NC, N_NODES, D), jnp.float32),
    mesh=_MESH,
    scratch_types=[
        pltpu.VMEM((NCH, CHUNK), jnp.int32),
        pltpu.VMEM((NCH, CHUNK), jnp.int32),
        pltpu.VMEM((CHUNK, D), jnp.float32),
        pltpu.VMEM_SHARED((N_NODES, D), jnp.float32),
        pltpu.SemaphoreType.DMA,
    ],
)(_sc_body)


def _combine_body(act, p_ref, xr_ref, b_ref, o_ref):
    o_ref[...] = act(p_ref[0] + p_ref[1] + xr_ref[0] + b_ref[0][None, :])


def _tc_combine(p, hall, b, act, nb=5):
    blk = N_NODES // nb
    return pl.pallas_call(
        functools.partial(_combine_body, act),
        grid=(nb,),
        in_specs=[
            pl.BlockSpec((NC, blk, D), lambda i: (0, i, 0)),
            pl.BlockSpec((1, blk, D), lambda i: (R_CAT - 1, i, 0)),
            pl.BlockSpec((1, D), lambda i: (0, 0)),
        ],
        out_specs=pl.BlockSpec((blk, D), lambda i: (i, 0)),
        out_shape=jax.ShapeDtypeStruct((N_NODES, D), jnp.float32),
    )(p, hall, b)


def _layer(x, wcat, b, gidx, dsti, zeros, act):
    hall = _tc_matmul(x, wcat)
    p = _sc_aggregate(hall.reshape(R_CAT * N_NODES, D), gidx, dsti, zeros)
    return _tc_combine(p, hall, b.reshape(1, D), act)


def kernel(x, edge_index, edge_type, W1, root1, b1, W2, root2, b2):
    src = edge_index[0].astype(jnp.int32)
    dst = edge_index[1].astype(jnp.int32)
    et = edge_type.astype(jnp.int32)
    gidx = (et * N_NODES + src).reshape(NW, NCH, CHUNK)
    dsti = dst.reshape(NW, NCH, CHUNK)
    zeros = jnp.zeros((N_NODES, D), jnp.float32)
    wcat1 = jnp.concatenate([W1, root1[None]], axis=0)
    wcat2 = jnp.concatenate([W2, root2[None]], axis=0)
    h = _layer(x, wcat1, b1, gidx, dsti, zeros,
               lambda v: jnp.maximum(v, 0.0))
    return _layer(h, wcat2, b2, gidx, dsti, zeros, jax.nn.sigmoid)


# speedup vs baseline: 27.3922x; 1.2876x over previous
import functools

import jax
import jax.numpy as jnp
from jax import lax
from jax.experimental import pallas as pl
from jax.experimental.pallas import tpu as pltpu
from jax.experimental.pallas import tpu_sc as plsc

N_NODES = 10000
D = 128
N_REL = 8
E = 320000
R_CAT = N_REL + 1

NC, NS = 2, 16
NW = NC * NS
EPW = E // NW
CHUNK = 125
NCH = EPW // CHUNK
NBUF = 2
NGRP = NCH // NBUF
ROWS_PT = 624
ROWS_TAIL = N_NODES - NS * ROWS_PT

_MESH = plsc.VectorSubcoreMesh(core_axis_name="c", subcore_axis_name="s")


def _mm_body(x_ref, w_ref, o_ref):
    o_ref[0] = jnp.dot(x_ref[...], w_ref[0], preferred_element_type=jnp.float32)


def _tc_matmul(x, wcat, nb=5):
    blk = N_NODES // nb
    return pl.pallas_call(
        _mm_body,
        grid=(R_CAT, nb),
        in_specs=[
            pl.BlockSpec((blk, D), lambda r, b: (b, 0)),
            pl.BlockSpec((1, D, D), lambda r, b: (r, 0, 0)),
        ],
        out_specs=pl.BlockSpec((1, blk, D), lambda r, b: (r, b, 0)),
        out_shape=jax.ShapeDtypeStruct((R_CAT, N_NODES, D), jnp.float32),
    )(x, wcat)


def _sc_body(hall, gidx_hbm, dst_hbm, zeros_hbm, out_hbm,
             ig_v, id_v, rows_v, agg_sp, igsems, idsems, gsems, ssems):
    c = lax.axis_index("c")
    s = lax.axis_index("s")
    wid = s * NC + c

    pltpu.sync_copy(zeros_hbm.at[pl.ds(s * ROWS_PT, ROWS_PT)],
                    agg_sp.at[pl.ds(s * ROWS_PT, ROWS_PT)])

    @pl.when(s == 0)
    def _init_tail():
        pltpu.sync_copy(zeros_hbm.at[pl.ds(NS * ROWS_PT, ROWS_TAIL)],
                        agg_sp.at[pl.ds(NS * ROWS_PT, ROWS_TAIL)])

    plsc.subcore_barrier()

    def _idx_start(g, slot):
        pltpu.async_copy(gidx_hbm.at[wid, g], ig_v.at[slot], igsems[slot])
        pltpu.async_copy(dst_hbm.at[wid, g], id_v.at[slot], idsems[slot])

    def _idx_wait(g, slot):
        pltpu.make_async_copy(gidx_hbm.at[wid, g], ig_v.at[slot],
                              igsems[slot]).wait()
        pltpu.make_async_copy(dst_hbm.at[wid, g], id_v.at[slot],
                              idsems[slot]).wait()

    def _start_gather(slot, b):
        pltpu.async_copy(hall.at[ig_v.at[slot, b]], rows_v.at[b], gsems[b])

    def _wait_gather(slot, b):
        pltpu.make_async_copy(hall.at[ig_v.at[slot, b]], rows_v.at[b],
                              gsems[b]).wait()

    def _start_scatter(slot, b):
        pltpu.async_copy(rows_v.at[b], agg_sp.at[id_v.at[slot, b]],
                         ssems[b], add=True)

    def _wait_scatter(slot, b):
        pltpu.make_async_copy(rows_v.at[b], agg_sp.at[id_v.at[slot, b]],
                              ssems[b]).wait()

    _idx_start(0, 0)
    _idx_start(1, 1)
    _idx_wait(0, 0)
    for b in range(NBUF):
        _start_gather(0, b)

    def pair(i, carry):
        for slot in range(2):
            g = 2 * i + slot
            for b in range(NBUF):
                _wait_gather(slot, b)
                _start_scatter(slot, b)

            @pl.when(g + 2 < NGRP)
            def _prefetch():
                _idx_start(g + 2, slot)

            @pl.when(g + 1 < NGRP)
            def _next_gathers():
                _idx_wait(g + 1, 1 - slot)
                for b in range(NBUF):
                    _wait_scatter(slot, b)
                    _start_gather(1 - slot, b)

        return carry

    lax.fori_loop(0, NGRP // 2, pair, 0)
    for b in range(NBUF):
        _wait_scatter(1, b)
    plsc.subcore_barrier()
    pltpu.sync_copy(agg_sp.at[pl.ds(s * ROWS_PT, ROWS_PT)],
                    out_hbm.at[c, pl.ds(s * ROWS_PT, ROWS_PT)])

    @pl.when(s == 0)
    def _write_tail():
        pltpu.sync_copy(agg_sp.at[pl.ds(NS * ROWS_PT, ROWS_TAIL)],
                        out_hbm.at[c, pl.ds(NS * ROWS_PT, ROWS_TAIL)])


_sc_aggregate = functools.partial(
    pl.kernel,
    out_type=jax.ShapeDtypeStruct((NC, N_NODES, D), jnp.float32),
    mesh=_MESH,
    scratch_types=[
        pltpu.VMEM((2, NBUF, CHUNK), jnp.int32),
        pltpu.VMEM((2, NBUF, CHUNK), jnp.int32),
        pltpu.VMEM((NBUF, CHUNK, D), jnp.float32),
        pltpu.VMEM_SHARED((N_NODES, D), jnp.float32),
        [pltpu.SemaphoreType.DMA] * 2,
        [pltpu.SemaphoreType.DMA] * 2,
        [pltpu.SemaphoreType.DMA] * NBUF,
        [pltpu.SemaphoreType.DMA] * NBUF,
    ],
)(_sc_body)


def _combine_body(act, p_ref, xr_ref, b_ref, o_ref):
    o_ref[...] = act(p_ref[0] + p_ref[1] + xr_ref[0] + b_ref[0][None, :])


def _tc_combine(p, hall, b, act, nb=5):
    blk = N_NODES // nb
    return pl.pallas_call(
        functools.partial(_combine_body, act),
        grid=(nb,),
        in_specs=[
            pl.BlockSpec((NC, blk, D), lambda i: (0, i, 0)),
            pl.BlockSpec((1, blk, D), lambda i: (R_CAT - 1, i, 0)),
            pl.BlockSpec((1, D), lambda i: (0, 0)),
        ],
        out_specs=pl.BlockSpec((blk, D), lambda i: (i, 0)),
        out_shape=jax.ShapeDtypeStruct((N_NODES, D), jnp.float32),
    )(p, hall, b)


def _layer(x, wcat, b, gidx, dsti, zeros, act):
    hall = _tc_matmul(x, wcat)
    p = _sc_aggregate(hall.reshape(R_CAT * N_NODES, D), gidx, dsti, zeros)
    return _tc_combine(p, hall, b.reshape(1, D), act)


def kernel(x, edge_index, edge_type, W1, root1, b1, W2, root2, b2):
    src = edge_index[0].astype(jnp.int32)
    dst = edge_index[1].astype(jnp.int32)
    et = edge_type.astype(jnp.int32)
    gidx = (et * N_NODES + src).reshape(NW, NGRP, NBUF, CHUNK)
    dsti = dst.reshape(NW, NGRP, NBUF, CHUNK)
    zeros = jnp.zeros((N_NODES, D), jnp.float32)
    wcat1 = jnp.concatenate([W1, root1[None]], axis=0)
    wcat2 = jnp.concatenate([W2, root2[None]], axis=0)
    h = _layer(x, wcat1, b1, gidx, dsti, zeros,
               lambda v: jnp.maximum(v, 0.0))
    return _layer(h, wcat2, b2, gidx, dsti, zeros, jax.nn.sigmoid)
